# baseline (speedup 1.0000x reference)
import functools

import jax
import jax.numpy as jnp
from jax import lax
from jax.experimental import pallas as pl
from jax.experimental.pallas import tpu as pltpu
from jax.experimental.pallas import tpu_sc as plsc

B = 4096
L = 50
EMB = 64
T = B * L

NC = 2
NS = 16
NW = NC * NS
TPW = T // NW
CHUNK = 128
NCHUNK = TPW // CHUNK


def _emb_kernel(seq_hbm, lab_hbm, tok_hbm, seg_hbm, out_hbm,
                idx_v, lab_v, rows, seg_rows, sem):
    wid = lax.axis_index("s") * NC + lax.axis_index("c")
    base = wid * TPW

    pltpu.sync_copy(seq_hbm.at[wid], idx_v)
    pltpu.sync_copy(lab_hbm.at[wid], lab_v)

    def chunk_body(j, carry):
        pltpu.async_copy(tok_hbm.at[idx_v.at[j]], rows, sem).wait()
        pltpu.async_copy(seg_hbm.at[lab_v.at[j]], seg_rows, sem).wait()

        def add_body(t, c):
            for q in range(EMB // 16):
                sl = pl.ds(q * 16, 16)
                rows[t, sl] = rows[t, sl] + seg_rows[t, sl]
            return c

        lax.fori_loop(0, CHUNK, add_body, 0, unroll=4)

        pltpu.sync_copy(rows, out_hbm.at[pl.ds(base + j * CHUNK, CHUNK)])
        return carry

    lax.fori_loop(0, NCHUNK, chunk_body, 0)


@jax.jit
def _emb(seq_flat, lab_flat, token_table, segment_table):
    mesh = plsc.VectorSubcoreMesh(core_axis_name="c", subcore_axis_name="s")
    run = pl.kernel(
        _emb_kernel,
        out_type=jax.ShapeDtypeStruct((T, EMB), jnp.float32),
        mesh=mesh,
        scratch_types=[
            pltpu.VMEM((NCHUNK, CHUNK), jnp.int32),
            pltpu.VMEM((NCHUNK, CHUNK), jnp.int32),
            pltpu.VMEM((CHUNK, EMB), jnp.float32),
            pltpu.VMEM((CHUNK, EMB), jnp.float32),
            pltpu.SemaphoreType.DMA,
        ],
        compiler_params=pltpu.CompilerParams(use_tc_tiling_on_sc=False),
    )
    return run(seq_flat, lab_flat, token_table, segment_table)


def kernel(seq, segment_label, token_table, segment_table):
    seq_w = seq.reshape(NW, NCHUNK, CHUNK).astype(jnp.int32)
    lab_w = segment_label.reshape(NW, NCHUNK, CHUNK).astype(jnp.int32)
    out = _emb(seq_w, lab_w, token_table, segment_table)
    return out.reshape(B, L, EMB)


# speedup vs baseline: 1.0002x; 1.0002x over previous
import jax
import jax.numpy as jnp
from jax import lax
from jax.experimental import pallas as pl
from jax.experimental.pallas import tpu as pltpu
from jax.experimental.pallas import tpu_sc as plsc

B = 4096
L = 50
EMB = 64
T = B * L

NC = 2
NS = 16
NW = NC * NS
TPW = T // NW
G = 128
NG = TPW // G
NBUF = 4


def _emb_kernel(seq_hbm, lab_hbm, tok_hbm, seg_hbm, out_hbm,
                idx_v, lab_v, rows, seg_rows, gsem, ssem):
    wid = lax.axis_index("s") * NC + lax.axis_index("c")
    base = wid * TPW

    pltpu.sync_copy(seq_hbm.at[wid], idx_v)
    pltpu.sync_copy(lab_hbm.at[wid], lab_v)

    def gathers(g, b):
        return (
            pltpu.make_async_copy(tok_hbm.at[idx_v.at[g]], rows.at[b], gsem[b]),
            pltpu.make_async_copy(seg_hbm.at[lab_v.at[g]], seg_rows.at[b], gsem[b]),
        )

    def store(g, b):
        return pltpu.make_async_copy(
            rows.at[b], out_hbm.at[pl.ds(base + g * G, G)], ssem[b])

    def start_gathers(g, b):
        for d in gathers(g, b):
            d.start()

    def wait_gathers(g, b):
        for d in gathers(g, b):
            d.wait()

    def consume(g, b):
        wait_gathers(g, b)

        def add_body(t, c):
            for q in range(EMB // 16):
                sl = pl.ds(q * 16, 16)
                plsc.addupdate(rows.at[b].at[t, sl], seg_rows[b, t, sl])
            return c

        lax.fori_loop(0, G, add_body, 0, unroll=4)
        store(g, b).start()

    start_gathers(0, 0)
    start_gathers(1, 1)
    start_gathers(2, 2)
    consume(0, 0)
    start_gathers(3, 3)
    consume(1, 1)
    store(0, 0).wait()
    start_gathers(4, 0)
    consume(2, 2)
    store(1, 1).wait()
    start_gathers(5, 1)
    consume(3, 3)

    def outer(o, carry):
        for i in range(NBUF):
            g = 4 + o * NBUF + i
            bpf = (i + 2) % NBUF
            store(g - 2, bpf).wait()
            start_gathers(g + 2, bpf)
            consume(g, i)
        return carry

    lax.fori_loop(0, (NG - 6) // NBUF, outer, 0)

    consume(NG - 2, (NG - 2) % NBUF)
    consume(NG - 1, (NG - 1) % NBUF)
    for g in range(NG - 4, NG):
        store(g, g % NBUF).wait()


@jax.jit
def _emb(seq_w, lab_w, token_table, segment_table):
    mesh = plsc.VectorSubcoreMesh(core_axis_name="c", subcore_axis_name="s")
    run = pl.kernel(
        _emb_kernel,
        out_type=jax.ShapeDtypeStruct((T, EMB), jnp.float32),
        mesh=mesh,
        scratch_types=[
            pltpu.VMEM((NG, G), jnp.int32),
            pltpu.VMEM((NG, G), jnp.int32),
            pltpu.VMEM((NBUF, G, EMB), jnp.float32),
            pltpu.VMEM((NBUF, G, EMB), jnp.float32),
            [pltpu.SemaphoreType.DMA] * NBUF,
            [pltpu.SemaphoreType.DMA] * NBUF,
        ],
        compiler_params=pltpu.CompilerParams(use_tc_tiling_on_sc=False),
    )
    return run(seq_w, lab_w, token_table, segment_table)


def kernel(seq, segment_label, token_table, segment_table):
    seq_w = seq.reshape(NW, NG, G).astype(jnp.int32)
    lab_w = segment_label.reshape(NW, NG, G).astype(jnp.int32)
    out = _emb(seq_w, lab_w, token_table, segment_table)
    return out.reshape(B, L, EMB)


# speedup vs baseline: 4.2515x; 4.2506x over previous
import jax
import jax.numpy as jnp
from jax import lax
from jax.experimental import pallas as pl
from jax.experimental.pallas import tpu as pltpu
from jax.experimental.pallas import tpu_sc as plsc

B = 4096
L = 50
EMB = 64
T = B * L

NC = 2
NS = 16
NW = NC * NS
TPW = T // NW
G = 128
NG = TPW // G
NBUF = 4
SEG_REP = 512


def _emb_kernel(seq_hbm, lab_hbm, tok_hbm, seg_hbm, out_hbm,
                idx_v, lab_v, rows, seg_rows, gsem, ssem):
    wid = lax.axis_index("s") * NC + lax.axis_index("c")
    base = wid * TPW

    pltpu.sync_copy(seq_hbm.at[wid], idx_v)
    pltpu.sync_copy(lab_hbm.at[wid], lab_v)

    def gathers(g, b):
        return (
            pltpu.make_async_copy(tok_hbm.at[idx_v.at[g]], rows.at[b], gsem[b]),
            pltpu.make_async_copy(seg_hbm.at[lab_v.at[g]], seg_rows.at[b], gsem[b]),
        )

    def store(g, b):
        return pltpu.make_async_copy(
            rows.at[b], out_hbm.at[pl.ds(base + g * G, G)], ssem[b])

    def start_gathers(g, b):
        for d in gathers(g, b):
            d.start()

    def consume(g, b):
        for d in gathers(g, b):
            d.wait()

        def add_body(t, c):
            for q in range(EMB // 16):
                sl = pl.ds(q * 16, 16)
                plsc.addupdate(rows.at[b].at[t, sl], seg_rows[b, t, sl])
            return c

        lax.fori_loop(0, G, add_body, 0, unroll=4)
        store(g, b).start()

    start_gathers(0, 0)
    start_gathers(1, 1)
    start_gathers(2, 2)
    consume(0, 0)
    start_gathers(3, 3)
    consume(1, 1)
    store(0, 0).wait()
    start_gathers(4, 0)
    consume(2, 2)
    store(1, 1).wait()
    start_gathers(5, 1)
    consume(3, 3)

    def outer(o, carry):
        for i in range(NBUF):
            g = 4 + o * NBUF + i
            bpf = (i + 2) % NBUF
            store(g - 2, bpf).wait()
            start_gathers(g + 2, bpf)
            consume(g, i)
        return carry

    lax.fori_loop(0, (NG - 6) // NBUF, outer, 0)

    consume(NG - 2, (NG - 2) % NBUF)
    consume(NG - 1, (NG - 1) % NBUF)
    for g in range(NG - 4, NG):
        store(g, g % NBUF).wait()


@jax.jit
def _emb(seq_w, segidx_w, token_table, seg_rep):
    mesh = plsc.VectorSubcoreMesh(core_axis_name="c", subcore_axis_name="s")
    run = pl.kernel(
        _emb_kernel,
        out_type=jax.ShapeDtypeStruct((T, EMB), jnp.float32),
        mesh=mesh,
        scratch_types=[
            pltpu.VMEM((NG, G), jnp.int32),
            pltpu.VMEM((NG, G), jnp.int32),
            pltpu.VMEM((NBUF, G, EMB), jnp.float32),
            pltpu.VMEM((NBUF, G, EMB), jnp.float32),
            [pltpu.SemaphoreType.DMA] * NBUF,
            [pltpu.SemaphoreType.DMA] * NBUF,
        ],
        compiler_params=pltpu.CompilerParams(use_tc_tiling_on_sc=False),
    )
    return run(seq_w, segidx_w, token_table, seg_rep)


def kernel(seq, segment_label, token_table, segment_table):
    seq_w = seq.reshape(NW, NG, G).astype(jnp.int32)
    seg_rep = jnp.tile(segment_table, (SEG_REP, 1))
    spread = (jnp.arange(T, dtype=jnp.int32) % SEG_REP) * 3
    segidx = segment_label.reshape(T).astype(jnp.int32) + spread
    segidx_w = segidx.reshape(NW, NG, G)
    out = _emb(seq_w, segidx_w, token_table, seg_rep)
    return out.reshape(B, L, EMB)
